# baseline (speedup 1.0000x reference)
import functools

import jax
import jax.numpy as jnp
from jax import lax
from jax.experimental import pallas as pl
from jax.experimental.pallas import tpu as pltpu
from jax.experimental.pallas import tpu_sc as plsc

H = 512
W = 512
B = 16
N = 2_000_000
NBINS = 2 * H * W * B

NC = 2
NS = 16
NW = NC * NS

CH_ROWS = 1600
VPC1 = CH_ROWS // 16
NCHUNK1 = N // CH_ROWS

SLOT_BINS = 65_536
HPAD = SLOT_BINS + 16
SPB = 8
NPASS = (B * SPB) // NW
CH2 = 16_384
VPC2 = CH2 // 16
IDX_LEN = N + CH2

_mesh = plsc.VectorSubcoreMesh(core_axis_name="c", subcore_axis_name="s")


@functools.partial(
    pl.kernel,
    out_type=(
        jax.ShapeDtypeStruct((NBINS,), jnp.float32),
        jax.ShapeDtypeStruct((NC * IDX_LEN,), jnp.int32),
    ),
    mesh=_mesh,
    scratch_types=[
        pltpu.VMEM((CH_ROWS * 5,), jnp.float32),
        pltpu.VMEM((HPAD,), jnp.float32),
        pltpu.VMEM((CH2,), jnp.int32),
        pltpu.VMEM((CH2,), jnp.int32),
        pltpu.VMEM((CH2 + 16,), jnp.int32),
        pltpu.VMEM((32,), jnp.int32),
        pltpu.SemaphoreType.DMA,
        pltpu.SemaphoreType.DMA,
    ],
    compiler_params=pltpu.CompilerParams(needs_layout_passes=False),
)
def _vox_kernel(ev_hbm, bnd_hbm, out_hbm, idx2_hbm, ev_v, hist_v, idx_a,
                idx_b, comp_v, bnd_v, sem_a, sem_b):
    c = lax.axis_index("c")
    s = lax.axis_index("s")
    pltpu.sync_copy(bnd_hbm, bnd_v)

    lane5 = lax.iota(jnp.int32, 16) * 5
    nch1 = (NCHUNK1 - s + NS - 1) // NS

    def a_chunk(i, _):
        cid = s + i * NS
        pltpu.sync_copy(ev_hbm.at[pl.ds(cid * CH_ROWS * 5, CH_ROWS * 5)],
                        ev_v)

        def a_vec(v, _):
            base = v * 80 + lane5
            x = plsc.load_gather(ev_v, [base]).astype(jnp.int32)
            y = plsc.load_gather(ev_v, [base + 1]).astype(jnp.int32)
            p = plsc.load_gather(ev_v, [base + 3]).astype(jnp.int32)
            b = plsc.load_gather(ev_v, [base + 4]).astype(jnp.int32)
            vi = x + y * W + p * (W * H) + b * (2 * W * H)
            comp_v[pl.ds(v * 16, 16)] = vi
            return 0

        lax.fori_loop(0, VPC1, a_vec, 0)
        pltpu.sync_copy(comp_v.at[pl.ds(0, CH_ROWS)],
                        idx2_hbm.at[pl.ds(c * IDX_LEN + cid * CH_ROWS,
                                          CH_ROWS)])
        return 0

    lax.fori_loop(0, nch1, a_chunk, 0)
    plsc.subcore_barrier()

    wid = s * NC + c
    one16 = jnp.full((16,), 1.0, jnp.float32)
    zero16 = jnp.zeros((16,), jnp.float32)
    sent16 = jnp.full((16,), SLOT_BINS, jnp.int32)
    top = jnp.uint32(SLOT_BINS)

    for p in range(NPASS):
        slot = p * NW + wid
        beta = slot // SPB
        bin_base = slot * SLOT_BINS
        bnd_vec = bnd_v[pl.ds(beta, 16)]
        lo_e = bnd_vec[0]
        hi_e = bnd_vec[1]
        base0 = (lo_e // 16) * 16
        n_vec = (hi_e - base0 + 15) // 16
        n_ch = (n_vec + VPC2 - 1) // VPC2

        @plsc.parallel_loop(0, HPAD // 16, unroll=8)
        def zero_body(j):
            hist_v[pl.ds(j * 16, 16)] = zero16

        def start(ci, buf, sem):
            bb = jnp.minimum(base0 + ci * CH2, N)
            pltpu.async_copy(idx2_hbm.at[pl.ds(c * IDX_LEN + bb, CH2)],
                             buf, sem)

        def drain(buf, sem):
            pltpu.make_async_copy(idx2_hbm.at[pl.ds(0, CH2)], buf, sem).wait()

        def process(ci, buf):
            nv = jnp.clip(n_vec - ci * VPC2, 0, VPC2)

            def p1_body(v, off):
                iv = buf[pl.ds(v * 16, 16)]
                rel = plsc.bitcast(iv - bin_base, jnp.uint32)
                m = rel < top
                plsc.store_compressed(comp_v.at[pl.ds(off, 16)],
                                      plsc.bitcast(rel, jnp.int32), mask=m)
                cnt = plsc.all_reduce_population_count(m)[0]
                return off + cnt

            off = lax.fori_loop(0, nv, p1_body, jnp.int32(0))
            comp_v[pl.ds(off, 16)] = sent16
            n2 = (off + 15) // 16

            @plsc.parallel_loop(0, n2, unroll=4)
            def p2_body(v):
                rv = comp_v[pl.ds(v * 16, 16)]
                plsc.addupdate_scatter(hist_v, [rv], one16)

        start(0, idx_a, sem_a)
        n_pair = (n_ch + 1) // 2

        def pair_body(g, _):
            c0 = 2 * g
            start(c0 + 1, idx_b, sem_b)
            drain(idx_a, sem_a)
            process(c0, idx_a)
            start(c0 + 2, idx_a, sem_a)
            drain(idx_b, sem_b)
            process(c0 + 1, idx_b)
            return 0

        lax.fori_loop(0, n_pair, pair_body, 0)
        drain(idx_a, sem_a)
        pltpu.sync_copy(hist_v.at[pl.ds(0, SLOT_BINS)],
                        out_hbm.at[pl.ds(bin_base, SLOT_BINS)])


@jax.jit
def kernel(events):
    bcol = events[:, 4]
    cuts = jnp.sum(
        (bcol[:, None] < jnp.arange(1, B, dtype=bcol.dtype)[None, :])
        .astype(jnp.int32),
        axis=0,
    )
    bnd = jnp.concatenate([
        jnp.zeros((1,), jnp.int32),
        cuts,
        jnp.full((32 - B,), N, jnp.int32),
    ])
    vox, _ = _vox_kernel(events.reshape(-1), bnd)
    return vox.reshape(-1, 2, H, W)


# speedup vs baseline: 1.0091x; 1.0091x over previous
import functools

import jax
import jax.numpy as jnp
from jax import lax
from jax.experimental import pallas as pl
from jax.experimental.pallas import tpu as pltpu
from jax.experimental.pallas import tpu_sc as plsc

H = 512
W = 512
B = 16
N = 2_000_000
NBINS = 2 * H * W * B

NC = 2
NS = 16
NW = NC * NS

CH_ROWS = 1600
VPC1 = CH_ROWS // 16
NCHUNK1 = N // CH_ROWS

SLOT_BINS = 65_536
HPAD = SLOT_BINS + 16
SPB = 8
NPASS = (B * SPB) // NW
CH2 = 16_384
VPC2 = CH2 // 16
IDX_LEN = N + CH2

_mesh = plsc.VectorSubcoreMesh(core_axis_name="c", subcore_axis_name="s")


@functools.partial(
    pl.kernel,
    out_type=(
        jax.ShapeDtypeStruct((NBINS,), jnp.float32),
        jax.ShapeDtypeStruct((NC * IDX_LEN,), jnp.int32),
    ),
    mesh=_mesh,
    scratch_types=[
        pltpu.VMEM((CH_ROWS * 5,), jnp.float32),
        pltpu.VMEM((HPAD,), jnp.float32),
        pltpu.VMEM((CH2,), jnp.int32),
        pltpu.VMEM((CH2,), jnp.int32),
        pltpu.VMEM((CH2 + 16,), jnp.int32),
        pltpu.VMEM((32,), jnp.int32),
        pltpu.SemaphoreType.DMA,
        pltpu.SemaphoreType.DMA,
    ],
    compiler_params=pltpu.CompilerParams(needs_layout_passes=False),
)
def _vox_kernel(ev_hbm, bnd_hbm, out_hbm, idx2_hbm, ev_v, hist_v, idx_a,
                idx_b, comp_v, bnd_v, sem_a, sem_b):
    c = lax.axis_index("c")
    s = lax.axis_index("s")
    pltpu.sync_copy(bnd_hbm, bnd_v)

    nch1 = (NCHUNK1 - s + NS - 1) // NS

    def a_chunk(i, _):
        cid = s + i * NS
        r0 = cid * CH_ROWS
        pltpu.sync_copy(ev_hbm.at[pl.ds(0 * N + r0, CH_ROWS)],
                        ev_v.at[pl.ds(0, CH_ROWS)])
        pltpu.sync_copy(ev_hbm.at[pl.ds(1 * N + r0, CH_ROWS)],
                        ev_v.at[pl.ds(CH_ROWS, CH_ROWS)])
        pltpu.sync_copy(ev_hbm.at[pl.ds(3 * N + r0, CH_ROWS)],
                        ev_v.at[pl.ds(2 * CH_ROWS, CH_ROWS)])
        pltpu.sync_copy(ev_hbm.at[pl.ds(4 * N + r0, CH_ROWS)],
                        ev_v.at[pl.ds(3 * CH_ROWS, CH_ROWS)])

        def a_vec(v, _):
            x = ev_v[pl.ds(v * 16, 16)].astype(jnp.int32)
            y = ev_v[pl.ds(CH_ROWS + v * 16, 16)].astype(jnp.int32)
            p = ev_v[pl.ds(2 * CH_ROWS + v * 16, 16)].astype(jnp.int32)
            b = ev_v[pl.ds(3 * CH_ROWS + v * 16, 16)].astype(jnp.int32)
            vi = x + y * W + p * (W * H) + b * (2 * W * H)
            comp_v[pl.ds(v * 16, 16)] = vi
            return 0

        lax.fori_loop(0, VPC1, a_vec, 0)
        pltpu.sync_copy(comp_v.at[pl.ds(0, CH_ROWS)],
                        idx2_hbm.at[pl.ds(c * IDX_LEN + cid * CH_ROWS,
                                          CH_ROWS)])
        return 0

    lax.fori_loop(0, nch1, a_chunk, 0)
    plsc.subcore_barrier()

    wid = s * NC + c
    one16 = jnp.full((16,), 1.0, jnp.float32)
    zero16 = jnp.zeros((16,), jnp.float32)
    sent16 = jnp.full((16,), SLOT_BINS, jnp.int32)
    top = jnp.uint32(SLOT_BINS)

    for p in range(NPASS):
        slot = p * NW + wid
        beta = slot // SPB
        bin_base = slot * SLOT_BINS
        bnd_vec = bnd_v[pl.ds(beta, 16)]
        lo_e = bnd_vec[0]
        hi_e = bnd_vec[1]
        base0 = (lo_e // 16) * 16
        n_vec = (hi_e - base0 + 15) // 16
        n_ch = (n_vec + VPC2 - 1) // VPC2

        @plsc.parallel_loop(0, HPAD // 16, unroll=8)
        def zero_body(j):
            hist_v[pl.ds(j * 16, 16)] = zero16

        def start(ci, buf, sem):
            bb = jnp.minimum(base0 + ci * CH2, N)
            pltpu.async_copy(idx2_hbm.at[pl.ds(c * IDX_LEN + bb, CH2)],
                             buf, sem)

        def drain(buf, sem):
            pltpu.make_async_copy(idx2_hbm.at[pl.ds(0, CH2)], buf, sem).wait()

        def process(ci, buf):
            nv = jnp.clip(n_vec - ci * VPC2, 0, VPC2)

            def p1_body(v, off):
                iv = buf[pl.ds(v * 16, 16)]
                rel = plsc.bitcast(iv - bin_base, jnp.uint32)
                m = rel < top
                plsc.store_compressed(comp_v.at[pl.ds(off, 16)],
                                      plsc.bitcast(rel, jnp.int32), mask=m)
                cnt = plsc.all_reduce_population_count(m)[0]
                return off + cnt

            off = lax.fori_loop(0, nv, p1_body, jnp.int32(0))
            comp_v[pl.ds(off, 16)] = sent16
            n2 = (off + 15) // 16

            @plsc.parallel_loop(0, n2, unroll=4)
            def p2_body(v):
                rv = comp_v[pl.ds(v * 16, 16)]
                plsc.addupdate_scatter(hist_v, [rv], one16)

        start(0, idx_a, sem_a)
        n_pair = (n_ch + 1) // 2

        def pair_body(g, _):
            c0 = 2 * g
            start(c0 + 1, idx_b, sem_b)
            drain(idx_a, sem_a)
            process(c0, idx_a)
            start(c0 + 2, idx_a, sem_a)
            drain(idx_b, sem_b)
            process(c0 + 1, idx_b)
            return 0

        lax.fori_loop(0, n_pair, pair_body, 0)
        drain(idx_a, sem_a)
        pltpu.sync_copy(hist_v.at[pl.ds(0, SLOT_BINS)],
                        out_hbm.at[pl.ds(bin_base, SLOT_BINS)])


@jax.jit
def kernel(events):
    bcol = events[:, 4]
    cuts = jnp.sum(
        (bcol[:, None] < jnp.arange(1, B, dtype=bcol.dtype)[None, :])
        .astype(jnp.int32),
        axis=0,
    )
    bnd = jnp.concatenate([
        jnp.zeros((1,), jnp.int32),
        cuts,
        jnp.full((32 - B,), N, jnp.int32),
    ])
    vox, _ = _vox_kernel(events.T.reshape(-1), bnd)
    return vox.reshape(-1, 2, H, W)


# speedup vs baseline: 1.8372x; 1.8206x over previous
import functools

import jax
import jax.numpy as jnp
from jax import lax
from jax.experimental import pallas as pl
from jax.experimental.pallas import tpu as pltpu
from jax.experimental.pallas import tpu_sc as plsc

H = 512
W = 512
B = 16
N = 2_000_000
NBINS = 2 * H * W * B

NC = 2
NS = 16
NW = NC * NS

CH_ROWS = 1600
VPC1 = CH_ROWS // 16
NCHUNK1 = N // CH_ROWS

SLOT_BINS = 65_536
HPAD = SLOT_BINS + 16
SPB = 8
NPASS = (B * SPB) // NW
CH2 = 16_384
VPC2 = CH2 // 16
IDX_LEN = N + CH2

_mesh = plsc.VectorSubcoreMesh(core_axis_name="c", subcore_axis_name="s")


@functools.partial(
    pl.kernel,
    out_type=(
        jax.ShapeDtypeStruct((NBINS,), jnp.float32),
        jax.ShapeDtypeStruct((NC * IDX_LEN,), jnp.int32),
    ),
    mesh=_mesh,
    scratch_types=[
        pltpu.VMEM((CH_ROWS * 5,), jnp.float32),
        pltpu.VMEM((HPAD,), jnp.float32),
        pltpu.VMEM((CH2,), jnp.int32),
        pltpu.VMEM((CH2,), jnp.int32),
        pltpu.VMEM((CH2 + 16,), jnp.int32),
        pltpu.VMEM((32,), jnp.int32),
        pltpu.SemaphoreType.DMA,
        pltpu.SemaphoreType.DMA,
    ],
    compiler_params=pltpu.CompilerParams(needs_layout_passes=False),
)
def _vox_kernel(x_hbm, y_hbm, p_hbm, b_hbm, bnd_hbm, out_hbm, idx2_hbm,
                ev_v, hist_v, idx_a, idx_b, comp_v, bnd_v, sem_a, sem_b):
    c = lax.axis_index("c")
    s = lax.axis_index("s")
    pltpu.sync_copy(bnd_hbm, bnd_v)

    nch1 = (NCHUNK1 - s + NS - 1) // NS

    def a_chunk(i, _):
        cid = s + i * NS
        r0 = cid * CH_ROWS
        pltpu.sync_copy(x_hbm.at[pl.ds(r0, CH_ROWS)],
                        ev_v.at[pl.ds(0, CH_ROWS)])
        pltpu.sync_copy(y_hbm.at[pl.ds(r0, CH_ROWS)],
                        ev_v.at[pl.ds(CH_ROWS, CH_ROWS)])
        pltpu.sync_copy(p_hbm.at[pl.ds(r0, CH_ROWS)],
                        ev_v.at[pl.ds(2 * CH_ROWS, CH_ROWS)])
        pltpu.sync_copy(b_hbm.at[pl.ds(r0, CH_ROWS)],
                        ev_v.at[pl.ds(3 * CH_ROWS, CH_ROWS)])

        def a_vec(v, _):
            x = ev_v[pl.ds(v * 16, 16)].astype(jnp.int32)
            y = ev_v[pl.ds(CH_ROWS + v * 16, 16)].astype(jnp.int32)
            p = ev_v[pl.ds(2 * CH_ROWS + v * 16, 16)].astype(jnp.int32)
            b = ev_v[pl.ds(3 * CH_ROWS + v * 16, 16)].astype(jnp.int32)
            vi = x + y * W + p * (W * H) + b * (2 * W * H)
            comp_v[pl.ds(v * 16, 16)] = vi
            return 0

        lax.fori_loop(0, VPC1, a_vec, 0)
        pltpu.sync_copy(comp_v.at[pl.ds(0, CH_ROWS)],
                        idx2_hbm.at[pl.ds(c * IDX_LEN + cid * CH_ROWS,
                                          CH_ROWS)])
        return 0

    lax.fori_loop(0, nch1, a_chunk, 0)
    plsc.subcore_barrier()

    wid = s * NC + c
    one16 = jnp.full((16,), 1.0, jnp.float32)
    zero16 = jnp.zeros((16,), jnp.float32)
    sent16 = jnp.full((16,), SLOT_BINS, jnp.int32)
    top = jnp.uint32(SLOT_BINS)

    for p in range(NPASS):
        slot = p * NW + wid
        beta = slot // SPB
        bin_base = slot * SLOT_BINS
        bnd_vec = bnd_v[pl.ds(beta, 16)]
        lo_e = bnd_vec[0]
        hi_e = bnd_vec[1]
        base0 = (lo_e // 16) * 16
        n_vec = (hi_e - base0 + 15) // 16
        n_ch = (n_vec + VPC2 - 1) // VPC2

        @plsc.parallel_loop(0, HPAD // 16, unroll=8)
        def zero_body(j):
            hist_v[pl.ds(j * 16, 16)] = zero16

        def start(ci, buf, sem):
            bb = jnp.minimum(base0 + ci * CH2, N)
            pltpu.async_copy(idx2_hbm.at[pl.ds(c * IDX_LEN + bb, CH2)],
                             buf, sem)

        def drain(buf, sem):
            pltpu.make_async_copy(idx2_hbm.at[pl.ds(0, CH2)], buf, sem).wait()

        def process(ci, buf):
            nv = jnp.clip(n_vec - ci * VPC2, 0, VPC2)

            def p1_body(v, off):
                iv = buf[pl.ds(v * 16, 16)]
                rel = plsc.bitcast(iv - bin_base, jnp.uint32)
                m = rel < top
                plsc.store_compressed(comp_v.at[pl.ds(off, 16)],
                                      plsc.bitcast(rel, jnp.int32), mask=m)
                cnt = plsc.all_reduce_population_count(m)[0]
                return off + cnt

            off = lax.fori_loop(0, nv, p1_body, jnp.int32(0))
            comp_v[pl.ds(off, 16)] = sent16
            n2 = (off + 15) // 16

            @plsc.parallel_loop(0, n2, unroll=4)
            def p2_body(v):
                rv = comp_v[pl.ds(v * 16, 16)]
                plsc.addupdate_scatter(hist_v, [rv], one16)

        start(0, idx_a, sem_a)
        n_pair = (n_ch + 1) // 2

        def pair_body(g, _):
            c0 = 2 * g
            start(c0 + 1, idx_b, sem_b)
            drain(idx_a, sem_a)
            process(c0, idx_a)
            start(c0 + 2, idx_a, sem_a)
            drain(idx_b, sem_b)
            process(c0 + 1, idx_b)
            return 0

        lax.fori_loop(0, n_pair, pair_body, 0)
        drain(idx_a, sem_a)
        pltpu.sync_copy(hist_v.at[pl.ds(0, SLOT_BINS)],
                        out_hbm.at[pl.ds(bin_base, SLOT_BINS)])


@jax.jit
def kernel(events):
    bcol = events[:, 4]
    cuts = jnp.sum(
        (bcol[:, None] < jnp.arange(1, B, dtype=bcol.dtype)[None, :])
        .astype(jnp.int32),
        axis=0,
    )
    bnd = jnp.concatenate([
        jnp.zeros((1,), jnp.int32),
        cuts,
        jnp.full((32 - B,), N, jnp.int32),
    ])
    vox, _ = _vox_kernel(events[:, 0], events[:, 1], events[:, 3],
                         events[:, 4], bnd)
    return vox.reshape(-1, 2, H, W)
